# baseline (speedup 1.0000x reference)
import functools

import jax
import jax.numpy as jnp
from jax import lax
from jax.experimental import pallas as pl
from jax.experimental.pallas import tpu as pltpu
from jax.experimental.pallas import tpu_sc as plsc

N = 131072
D = 190
NW = 32
CH = N // NW
NCH = CH // 16
GB = 128
NG = CH // GB

_mesh = plsc.VectorSubcoreMesh(core_axis_name="c", subcore_axis_name="s")
_params = pltpu.CompilerParams(needs_layout_passes=False,
                               use_tc_tiling_on_sc=False)


SLAB = 256
NSLAB = CH // SLAB
SCHK = SLAB // 16


@functools.partial(
    pl.kernel,
    out_type=(
        jax.ShapeDtypeStruct((N,), jnp.float32),
        jax.ShapeDtypeStruct((N,), jnp.int32),
        jax.ShapeDtypeStruct((N,), jnp.int32),
        jax.ShapeDtypeStruct((NW * 32,), jnp.int32),
    ),
    mesh=_mesh,
    compiler_params=_params,
    scratch_types=[
        pltpu.VMEM((CH,), jnp.int32),
        pltpu.VMEM((SLAB, D), jnp.float32),
        pltpu.VMEM((SLAB, D), jnp.float32),
        pltpu.VMEM((CH,), jnp.float32),
        pltpu.VMEM((CH,), jnp.int32),
        pltpu.VMEM((CH,), jnp.int32),
        pltpu.VMEM((32,), jnp.int32),
        pltpu.SemaphoreType.DMA,
        pltpu.SemaphoreType.DMA,
    ],
)
def _scan_and_gather(tab_hbm, ct_hbm, pred_hbm, aux1_hbm, aux2_hbm, hist_hbm,
                     ct_v, slab0_v, slab1_v, pred_v, aux1_v, aux2_v, cnt_v,
                     sem0, sem1):
    wid = lax.axis_index("s") * 2 + lax.axis_index("c")
    base = wid * CH
    pltpu.sync_copy(ct_hbm.at[pl.ds(base, CH)], ct_v)

    zeros16 = jnp.zeros((16,), jnp.int32)
    cnt_v[0:16] = zeros16
    cnt_v[16:32] = zeros16
    iota = lax.iota(jnp.int32, 16)
    c17i = jnp.full((16,), 17, jnp.int32)
    c18i = jnp.full((16,), 18, jnp.int32)
    slabs = (slab0_v, slab1_v)
    sems = (sem0, sem1)

    def start(s):
        return pltpu.async_copy(
            tab_hbm.at[pl.ds(base + s * SLAB, SLAB), :], slabs[s % 2],
            sems[s % 2],
        )

    pending = start(0)
    for s in range(NSLAB):
        pending.wait()
        if s + 1 < NSLAB:
            pending = start(s + 1)
        slab = slabs[s % 2]

        def body(j2, carry, s=s, slab=slab):
            j = s * SCHK + j2
            j16 = j * 16
            v = ct_v[pl.ds(j16, 16)]
            row = j2 * 16 + iota
            pred_v[pl.ds(j16, 16)] = plsc.load_gather(slab, [row, v])
            m9 = (v % 10) == 9
            cls = v // 10
            cnt, lastm = plsc.scan_count(v, m9)
            old = plsc.load_gather(cnt_v, [cls], mask=m9)
            aux1_v[pl.ds(j16, 16)] = jnp.where(m9, old + cnt - 1, 0)
            c17c = plsc.load_gather(cnt_v, [c17i])
            c18c = plsc.load_gather(cnt_v, [c18i])
            plsc.store_scatter(cnt_v, [cls], old + cnt, mask=lastm)
            eq189 = v == 189
            eq179 = v == 179
            i189 = eq189.astype(jnp.int32)
            i179 = eq179.astype(jnp.int32)
            cs = plsc.cumsum(i189 + i179 * 65536)
            e18 = (cs & 0xFFFF) - i189
            e17 = (cs >> 16) - i179
            aux2_v[pl.ds(j16, 16)] = jnp.where(eq189, c17c + e17, c18c + e18)
            return carry

        lax.fori_loop(0, SCHK, body, 0)

    pltpu.sync_copy(pred_v, pred_hbm.at[pl.ds(base, CH)])
    pltpu.sync_copy(aux1_v, aux1_hbm.at[pl.ds(base, CH)])
    pltpu.sync_copy(aux2_v, aux2_hbm.at[pl.ds(base, CH)])
    pltpu.sync_copy(cnt_v, hist_hbm.at[pl.ds(wid * 32, 32)])


@functools.partial(
    pl.kernel,
    out_type=(
        jax.ShapeDtypeStruct((N,), jnp.float32),
        jax.ShapeDtypeStruct((N,), jnp.float32),
    ),
    mesh=_mesh,
    compiler_params=_params,
    scratch_types=[
        pltpu.VMEM((CH,), jnp.int32),
        pltpu.VMEM((CH,), jnp.int32),
        pltpu.VMEM((CH,), jnp.int32),
        pltpu.VMEM((NW * 32,), jnp.int32),
        pltpu.VMEM((32,), jnp.int32),
        pltpu.VMEM((CH,), jnp.int32),
        pltpu.VMEM((CH,), jnp.int32),
        pltpu.VMEM((CH,), jnp.float32),
        pltpu.VMEM((CH,), jnp.float32),
        pltpu.VMEM((CH,), jnp.float32),
        pltpu.SemaphoreType.DMA,
    ],
)
def _resolve(pred_hbm, ct_hbm, aux1_hbm, aux2_hbm, hist_hbm, edge_hbm, body_hbm,
             ct_v, aux1_v, aux2_v, hist_v, offs_v, idxe_v, idxb_v, ev_v, bv_v,
             eo_v, sem):
    wid = lax.axis_index("s") * 2 + lax.axis_index("c")
    base = wid * CH
    pltpu.sync_copy(ct_hbm.at[pl.ds(base, CH)], ct_v)
    pltpu.sync_copy(aux1_hbm.at[pl.ds(base, CH)], aux1_v)
    pltpu.sync_copy(aux2_hbm.at[pl.ds(base, CH)], aux2_v)
    pltpu.sync_copy(hist_hbm, hist_v)

    zeros16 = jnp.zeros((16,), jnp.int32)

    def hbody(t, carry):
        lo, hi = carry
        add = t < wid
        row_lo = hist_v[pl.ds(t * 32, 16)]
        row_hi = hist_v[pl.ds(t * 32 + 16, 16)]
        lo = lo + jnp.where(add, row_lo, zeros16)
        hi = hi + jnp.where(add, row_hi, zeros16)
        return lo, hi

    lo, hi = lax.fori_loop(0, NW, hbody, (zeros16, zeros16))
    offs_v[0:16] = lo
    offs_v[16:32] = hi

    iota = lax.iota(jnp.int32, 16)
    p17 = plsc.load_gather(offs_v, [jnp.full((16,), 17, jnp.int32)])
    p18 = plsc.load_gather(offs_v, [jnp.full((16,), 18, jnp.int32)])

    def pbody(j, carry):
        j16 = j * 16
        idxe_v[pl.ds(j16, 16)] = j16 + iota
        return carry

    lax.fori_loop(0, NCH, pbody, 0)

    def body(j, off):
        j16 = j * 16
        v = ct_v[pl.ds(j16, 16)]
        m9 = (v % 10) == 9
        cls = v // 10
        goff = plsc.load_gather(offs_v, [cls], mask=m9)
        plsc.store_compressed(
            idxe_v.at[pl.ds(off, 16)], goff + aux1_v[pl.ds(j16, 16)], mask=m9
        )
        n = base + j16 + iota
        idxb_v[pl.ds(j16, 16)] = (
            n - jnp.where(v == 189, p17, p18) - aux2_v[pl.ds(j16, 16)]
        )
        return off + jnp.sum(m9.astype(jnp.int32))

    lax.fori_loop(0, NCH, body, jnp.int32(0))

    copies = [
        pltpu.async_copy(
            pred_hbm.at[idxe_v.at[pl.ds(r * GB, GB)]],
            ev_v.at[pl.ds(r * GB, GB)],
            sem,
        )
        for r in range(NG)
    ] + [
        pltpu.async_copy(
            pred_hbm.at[idxb_v.at[pl.ds(r * GB, GB)]],
            bv_v.at[pl.ds(r * GB, GB)],
            sem,
        )
        for r in range(NG)
    ]
    for c in copies:
        c.wait()

    def mbody(j, off):
        j16 = j * 16
        v = ct_v[pl.ds(j16, 16)]
        m9 = (v % 10) == 9
        e = plsc.load_expanded(ev_v.at[pl.ds(off, 16)], mask=m9)
        eo_v[pl.ds(j16, 16)] = jnp.where(m9, e, 0.0)
        return off + jnp.sum(m9.astype(jnp.int32))

    lax.fori_loop(0, NCH, mbody, jnp.int32(0))

    pltpu.sync_copy(eo_v, edge_hbm.at[pl.ds(base, CH)])
    pltpu.sync_copy(bv_v, body_hbm.at[pl.ds(base, CH)])


def kernel(seg_edge, seg_body, contrast_logits, contrast_target, target,
           gt_boundary):
    del seg_edge, seg_body, target, gt_boundary
    ct = contrast_target.astype(jnp.int32)
    pred, aux1, aux2, hist = _scan_and_gather(contrast_logits, ct)
    edge, body = _resolve(pred, ct, aux1, aux2, hist)
    return edge, body


# speedup vs baseline: 1.0139x; 1.0139x over previous
import functools

import jax
import jax.numpy as jnp
from jax import lax
from jax.experimental import pallas as pl
from jax.experimental.pallas import tpu as pltpu
from jax.experimental.pallas import tpu_sc as plsc

N = 131072
D = 190
NW = 32
CH = N // NW
NCH = CH // 16
GB = 128
NG = CH // GB

_mesh = plsc.VectorSubcoreMesh(core_axis_name="c", subcore_axis_name="s")
_params = pltpu.CompilerParams(needs_layout_passes=False,
                               use_tc_tiling_on_sc=False)


TBLK = 512


def _pred_block(logits_ref, ct_ref, pred_ref):
    cols = lax.broadcasted_iota(jnp.int32, (TBLK, D), 1)
    sel = cols == ct_ref[...]
    pred_ref[...] = jnp.sum(
        jnp.where(sel, logits_ref[...], 0.0), axis=1, keepdims=True
    )


_pred_gather_tc = pl.pallas_call(
    _pred_block,
    grid=(N // TBLK,),
    in_specs=[
        pl.BlockSpec((TBLK, D), lambda i: (i, 0)),
        pl.BlockSpec((TBLK, 1), lambda i: (i, 0)),
    ],
    out_specs=pl.BlockSpec((TBLK, 1), lambda i: (i, 0)),
    out_shape=jax.ShapeDtypeStruct((N, 1), jnp.float32),
)


@functools.partial(
    pl.kernel,
    out_type=(
        jax.ShapeDtypeStruct((N,), jnp.int32),
        jax.ShapeDtypeStruct((N,), jnp.int32),
        jax.ShapeDtypeStruct((NW * 32,), jnp.int32),
    ),
    mesh=_mesh,
    compiler_params=_params,
    scratch_types=[
        pltpu.VMEM((CH,), jnp.int32),
        pltpu.VMEM((CH,), jnp.int32),
        pltpu.VMEM((CH,), jnp.int32),
        pltpu.VMEM((32,), jnp.int32),
    ],
)
def _scan_ranks(ct_hbm, aux1_hbm, aux2_hbm, hist_hbm,
                ct_v, aux1_v, aux2_v, cnt_v):
    wid = lax.axis_index("s") * 2 + lax.axis_index("c")
    base = wid * CH
    pltpu.sync_copy(ct_hbm.at[pl.ds(base, CH)], ct_v)

    zeros16 = jnp.zeros((16,), jnp.int32)
    cnt_v[0:16] = zeros16
    cnt_v[16:32] = zeros16
    c17i = jnp.full((16,), 17, jnp.int32)
    c18i = jnp.full((16,), 18, jnp.int32)

    def body(j, carry):
        j16 = j * 16
        v = ct_v[pl.ds(j16, 16)]
        m9 = (v % 10) == 9
        cls = v // 10
        cnt, lastm = plsc.scan_count(v, m9)
        old = plsc.load_gather(cnt_v, [cls], mask=m9)
        aux1_v[pl.ds(j16, 16)] = jnp.where(m9, old + cnt - 1, 0)
        c17c = plsc.load_gather(cnt_v, [c17i])
        c18c = plsc.load_gather(cnt_v, [c18i])
        plsc.store_scatter(cnt_v, [cls], old + cnt, mask=lastm)
        eq189 = v == 189
        eq179 = v == 179
        i189 = eq189.astype(jnp.int32)
        i179 = eq179.astype(jnp.int32)
        cs = plsc.cumsum(i189 + i179 * 65536)
        e18 = (cs & 0xFFFF) - i189
        e17 = (cs >> 16) - i179
        aux2_v[pl.ds(j16, 16)] = jnp.where(eq189, c17c + e17, c18c + e18)
        return carry

    lax.fori_loop(0, NCH, body, 0)

    pltpu.sync_copy(aux1_v, aux1_hbm.at[pl.ds(base, CH)])
    pltpu.sync_copy(aux2_v, aux2_hbm.at[pl.ds(base, CH)])
    pltpu.sync_copy(cnt_v, hist_hbm.at[pl.ds(wid * 32, 32)])


@functools.partial(
    pl.kernel,
    out_type=(
        jax.ShapeDtypeStruct((N,), jnp.float32),
        jax.ShapeDtypeStruct((N,), jnp.float32),
    ),
    mesh=_mesh,
    compiler_params=_params,
    scratch_types=[
        pltpu.VMEM((CH,), jnp.int32),
        pltpu.VMEM((CH,), jnp.int32),
        pltpu.VMEM((CH,), jnp.int32),
        pltpu.VMEM((NW * 32,), jnp.int32),
        pltpu.VMEM((32,), jnp.int32),
        pltpu.VMEM((CH,), jnp.int32),
        pltpu.VMEM((CH,), jnp.int32),
        pltpu.VMEM((CH,), jnp.float32),
        pltpu.VMEM((CH,), jnp.float32),
        pltpu.VMEM((CH,), jnp.float32),
        pltpu.SemaphoreType.DMA,
    ],
)
def _resolve(pred_hbm, ct_hbm, aux1_hbm, aux2_hbm, hist_hbm, edge_hbm, body_hbm,
             ct_v, aux1_v, aux2_v, hist_v, offs_v, idxe_v, idxb_v, ev_v, bv_v,
             eo_v, sem):
    wid = lax.axis_index("s") * 2 + lax.axis_index("c")
    base = wid * CH
    pltpu.sync_copy(ct_hbm.at[pl.ds(base, CH)], ct_v)
    pltpu.sync_copy(aux1_hbm.at[pl.ds(base, CH)], aux1_v)
    pltpu.sync_copy(aux2_hbm.at[pl.ds(base, CH)], aux2_v)
    pltpu.sync_copy(hist_hbm, hist_v)

    zeros16 = jnp.zeros((16,), jnp.int32)

    def hbody(t, carry):
        lo, hi = carry
        add = t < wid
        row_lo = hist_v[pl.ds(t * 32, 16)]
        row_hi = hist_v[pl.ds(t * 32 + 16, 16)]
        lo = lo + jnp.where(add, row_lo, zeros16)
        hi = hi + jnp.where(add, row_hi, zeros16)
        return lo, hi

    lo, hi = lax.fori_loop(0, NW, hbody, (zeros16, zeros16))
    offs_v[0:16] = lo
    offs_v[16:32] = hi

    iota = lax.iota(jnp.int32, 16)
    p17 = plsc.load_gather(offs_v, [jnp.full((16,), 17, jnp.int32)])
    p18 = plsc.load_gather(offs_v, [jnp.full((16,), 18, jnp.int32)])

    def pbody(j, carry):
        j16 = j * 16
        idxe_v[pl.ds(j16, 16)] = j16 + iota
        return carry

    lax.fori_loop(0, NCH, pbody, 0)

    def body(j, off):
        j16 = j * 16
        v = ct_v[pl.ds(j16, 16)]
        m9 = (v % 10) == 9
        cls = v // 10
        goff = plsc.load_gather(offs_v, [cls], mask=m9)
        plsc.store_compressed(
            idxe_v.at[pl.ds(off, 16)], goff + aux1_v[pl.ds(j16, 16)], mask=m9
        )
        n = base + j16 + iota
        idxb_v[pl.ds(j16, 16)] = (
            n - jnp.where(v == 189, p17, p18) - aux2_v[pl.ds(j16, 16)]
        )
        return off + jnp.sum(m9.astype(jnp.int32))

    lax.fori_loop(0, NCH, body, jnp.int32(0))

    copies = [
        pltpu.async_copy(
            pred_hbm.at[idxe_v.at[pl.ds(r * GB, GB)]],
            ev_v.at[pl.ds(r * GB, GB)],
            sem,
        )
        for r in range(NG)
    ] + [
        pltpu.async_copy(
            pred_hbm.at[idxb_v.at[pl.ds(r * GB, GB)]],
            bv_v.at[pl.ds(r * GB, GB)],
            sem,
        )
        for r in range(NG)
    ]
    for c in copies:
        c.wait()

    def mbody(j, off):
        j16 = j * 16
        v = ct_v[pl.ds(j16, 16)]
        m9 = (v % 10) == 9
        e = plsc.load_expanded(ev_v.at[pl.ds(off, 16)], mask=m9)
        eo_v[pl.ds(j16, 16)] = jnp.where(m9, e, 0.0)
        return off + jnp.sum(m9.astype(jnp.int32))

    lax.fori_loop(0, NCH, mbody, jnp.int32(0))

    pltpu.sync_copy(eo_v, edge_hbm.at[pl.ds(base, CH)])
    pltpu.sync_copy(bv_v, body_hbm.at[pl.ds(base, CH)])


def kernel(seg_edge, seg_body, contrast_logits, contrast_target, target,
           gt_boundary):
    del seg_edge, seg_body, target, gt_boundary
    ct = contrast_target.astype(jnp.int32)
    pred = _pred_gather_tc(contrast_logits, ct.reshape(N, 1)).reshape(N)
    aux1, aux2, hist = _scan_ranks(ct)
    edge, body = _resolve(pred, ct, aux1, aux2, hist)
    return edge, body


# speedup vs baseline: 1.4010x; 1.3817x over previous
import functools

import jax
import jax.numpy as jnp
from jax import lax
from jax.experimental import pallas as pl
from jax.experimental.pallas import tpu as pltpu
from jax.experimental.pallas import tpu_sc as plsc

N = 131072
D = 190
NW = 32
CH = N // NW
NCH = CH // 16
GB = 128
NG = CH // GB

_mesh = plsc.VectorSubcoreMesh(core_axis_name="c", subcore_axis_name="s")
_params = pltpu.CompilerParams(needs_layout_passes=False,
                               use_tc_tiling_on_sc=False)


@functools.partial(
    pl.kernel,
    out_type=(
        jax.ShapeDtypeStruct((N,), jnp.float32),
        jax.ShapeDtypeStruct((N,), jnp.int32),
        jax.ShapeDtypeStruct((N,), jnp.int32),
        jax.ShapeDtypeStruct((NW * 32,), jnp.int32),
    ),
    mesh=_mesh,
    compiler_params=_params,
    scratch_types=[
        pltpu.VMEM((CH,), jnp.int32),
        pltpu.VMEM((CH,), jnp.int32),
        pltpu.VMEM((CH,), jnp.float32),
        pltpu.VMEM((CH,), jnp.int32),
        pltpu.VMEM((CH,), jnp.int32),
        pltpu.VMEM((32,), jnp.int32),
        pltpu.SemaphoreType.DMA,
    ],
)
def _scan_and_gather(tab_hbm, ct_hbm, pred_hbm, aux1_hbm, aux2_hbm, hist_hbm,
                     ct_v, fidx_v, pred_v, aux1_v, aux2_v, cnt_v, sem):
    wid = lax.axis_index("s") * 2 + lax.axis_index("c")
    base = wid * CH
    pltpu.sync_copy(ct_hbm.at[pl.ds(base, CH)], ct_v)
    iota = lax.iota(jnp.int32, 16)

    zeros16 = jnp.zeros((16,), jnp.int32)
    cnt_v[0:16] = zeros16
    cnt_v[16:32] = zeros16
    c17i = jnp.full((16,), 17, jnp.int32)
    c18i = jnp.full((16,), 18, jnp.int32)

    def body(j, carry):
        j16 = j * 16
        v = ct_v[pl.ds(j16, 16)]
        fidx_v[pl.ds(j16, 16)] = (base + j16 + iota) * D + v
        m9 = (v % 10) == 9
        cls = v // 10
        cnt, lastm = plsc.scan_count(v, m9)
        old = plsc.load_gather(cnt_v, [cls], mask=m9)
        aux1_v[pl.ds(j16, 16)] = jnp.where(m9, old + cnt - 1, 0)
        c17c = plsc.load_gather(cnt_v, [c17i])
        c18c = plsc.load_gather(cnt_v, [c18i])
        plsc.store_scatter(cnt_v, [cls], old + cnt, mask=lastm)
        eq189 = v == 189
        eq179 = v == 179
        i189 = eq189.astype(jnp.int32)
        i179 = eq179.astype(jnp.int32)
        cs = plsc.cumsum(i189 + i179 * 65536)
        e18 = (cs & 0xFFFF) - i189
        e17 = (cs >> 16) - i179
        aux2_v[pl.ds(j16, 16)] = jnp.where(eq189, c17c + e17, c18c + e18)
        return carry

    lax.fori_loop(0, NCH, body, 0)

    copies = [
        pltpu.async_copy(
            tab_hbm.at[fidx_v.at[pl.ds(r * GB, GB)]],
            pred_v.at[pl.ds(r * GB, GB)],
            sem,
        )
        for r in range(NG)
    ]
    for c in copies:
        c.wait()

    pltpu.sync_copy(pred_v, pred_hbm.at[pl.ds(base, CH)])
    pltpu.sync_copy(aux1_v, aux1_hbm.at[pl.ds(base, CH)])
    pltpu.sync_copy(aux2_v, aux2_hbm.at[pl.ds(base, CH)])
    pltpu.sync_copy(cnt_v, hist_hbm.at[pl.ds(wid * 32, 32)])


@functools.partial(
    pl.kernel,
    out_type=(
        jax.ShapeDtypeStruct((N,), jnp.float32),
        jax.ShapeDtypeStruct((N,), jnp.float32),
    ),
    mesh=_mesh,
    compiler_params=_params,
    scratch_types=[
        pltpu.VMEM((CH,), jnp.int32),
        pltpu.VMEM((CH,), jnp.int32),
        pltpu.VMEM((CH,), jnp.int32),
        pltpu.VMEM((NW * 32,), jnp.int32),
        pltpu.VMEM((32,), jnp.int32),
        pltpu.VMEM((CH,), jnp.int32),
        pltpu.VMEM((CH,), jnp.int32),
        pltpu.VMEM((CH,), jnp.float32),
        pltpu.VMEM((CH,), jnp.float32),
        pltpu.VMEM((CH,), jnp.float32),
        pltpu.SemaphoreType.DMA,
    ],
)
def _resolve(pred_hbm, ct_hbm, aux1_hbm, aux2_hbm, hist_hbm, edge_hbm, body_hbm,
             ct_v, aux1_v, aux2_v, hist_v, offs_v, idxe_v, idxb_v, ev_v, bv_v,
             eo_v, sem):
    wid = lax.axis_index("s") * 2 + lax.axis_index("c")
    base = wid * CH
    pltpu.sync_copy(ct_hbm.at[pl.ds(base, CH)], ct_v)
    pltpu.sync_copy(aux1_hbm.at[pl.ds(base, CH)], aux1_v)
    pltpu.sync_copy(aux2_hbm.at[pl.ds(base, CH)], aux2_v)
    pltpu.sync_copy(hist_hbm, hist_v)

    zeros16 = jnp.zeros((16,), jnp.int32)

    def hbody(t, carry):
        lo, hi = carry
        add = t < wid
        row_lo = hist_v[pl.ds(t * 32, 16)]
        row_hi = hist_v[pl.ds(t * 32 + 16, 16)]
        lo = lo + jnp.where(add, row_lo, zeros16)
        hi = hi + jnp.where(add, row_hi, zeros16)
        return lo, hi

    lo, hi = lax.fori_loop(0, NW, hbody, (zeros16, zeros16))
    offs_v[0:16] = lo
    offs_v[16:32] = hi

    iota = lax.iota(jnp.int32, 16)
    p17 = plsc.load_gather(offs_v, [jnp.full((16,), 17, jnp.int32)])
    p18 = plsc.load_gather(offs_v, [jnp.full((16,), 18, jnp.int32)])

    def pbody(j, carry):
        j16 = j * 16
        idxe_v[pl.ds(j16, 16)] = j16 + iota
        return carry

    lax.fori_loop(0, NCH, pbody, 0)

    def body(j, off):
        j16 = j * 16
        v = ct_v[pl.ds(j16, 16)]
        m9 = (v % 10) == 9
        cls = v // 10
        goff = plsc.load_gather(offs_v, [cls], mask=m9)
        plsc.store_compressed(
            idxe_v.at[pl.ds(off, 16)], goff + aux1_v[pl.ds(j16, 16)], mask=m9
        )
        n = base + j16 + iota
        idxb_v[pl.ds(j16, 16)] = (
            n - jnp.where(v == 189, p17, p18) - aux2_v[pl.ds(j16, 16)]
        )
        return off + jnp.sum(m9.astype(jnp.int32))

    ne = lax.fori_loop(0, NCH, body, jnp.int32(0))

    for r in range(NG):
        @pl.when(ne > r * GB)
        def _(r=r):
            pltpu.async_copy(
                pred_hbm.at[idxe_v.at[pl.ds(r * GB, GB)]],
                ev_v.at[pl.ds(r * GB, GB)],
                sem,
            )

    copies = [
        pltpu.async_copy(
            pred_hbm.at[idxb_v.at[pl.ds(r * GB, GB)]],
            bv_v.at[pl.ds(r * GB, GB)],
            sem,
        )
        for r in range(NG)
    ]
    for c in copies:
        c.wait()
    for r in range(NG):
        @pl.when(ne > r * GB)
        def _(r=r):
            pltpu.make_async_copy(
                pred_hbm.at[idxe_v.at[pl.ds(r * GB, GB)]],
                ev_v.at[pl.ds(r * GB, GB)],
                sem,
            ).wait()

    def mbody(j, off):
        j16 = j * 16
        v = ct_v[pl.ds(j16, 16)]
        m9 = (v % 10) == 9
        e = plsc.load_expanded(ev_v.at[pl.ds(off, 16)], mask=m9)
        eo_v[pl.ds(j16, 16)] = jnp.where(m9, e, 0.0)
        return off + jnp.sum(m9.astype(jnp.int32))

    lax.fori_loop(0, NCH, mbody, jnp.int32(0))

    pltpu.sync_copy(eo_v, edge_hbm.at[pl.ds(base, CH)])
    pltpu.sync_copy(bv_v, body_hbm.at[pl.ds(base, CH)])


def kernel(seg_edge, seg_body, contrast_logits, contrast_target, target,
           gt_boundary):
    del seg_edge, seg_body, target, gt_boundary
    ct = contrast_target.astype(jnp.int32)
    flat = contrast_logits.reshape(-1)
    pred, aux1, aux2, hist = _scan_and_gather(flat, ct)
    edge, body = _resolve(pred, ct, aux1, aux2, hist)
    return edge, body
